# baseline (speedup 1.0000x reference)
import functools

import numpy as np
import jax
import jax.numpy as jnp
from jax import lax
from jax.experimental import pallas as pl
from jax.experimental.pallas import tpu as pltpu
from jax.experimental.pallas import tpu_sc as plsc

NCLS = 10
NC = 2
NS = 16
NW = NC * NS
LANES = 16
UNROLL = 8
NROUNDS = 16
HB = 11
HBINS = 1 << HB
SIGNBIT = np.int32(-(1 << 31))


def _draw_terms(H):
    rng = np.random.default_rng(0)
    terms = []
    for _ in range(1, NCLS):
        _ = int(rng.integers(1, min(5, H) + 1))
        ts = int(rng.integers(1, min(3, H) + 1))
        sel = [int(v) for v in rng.integers(0, H, size=(ts,))]
        sg = [bool(v) for v in rng.integers(0, 2, size=(ts,))]
        terms.append((sel, sg))
    return terms


def _skey(bits):
    return bits ^ lax.shift_right_logical(
        lax.shift_right_arithmetic(bits, 31), 1)


def _med_body(T, H, d, k, xflat_hbm, colidx_hbm, med_out, xs_out,
              colbuf, hist, idx0, idx1, cvec, medbuf):
    wid = lax.axis_index("s") * NC + lax.axis_index("c")
    nchunks = T // LANES
    ones = jnp.ones((LANES,), jnp.int32)
    lanes = lax.iota(jnp.int32, LANES)
    laneoff = lanes * jnp.int32(HBINS)
    dflip = jnp.int32(1 << (HB - 1))
    dmask = jnp.int32(HBINS - 1)

    pltpu.sync_copy(colidx_hbm, cvec)
    cv = cvec[pl.ds((wid // jnp.int32(LANES)) * LANES, LANES)]
    c = jnp.sum(jnp.where(lanes == (wid % jnp.int32(LANES)), cv, 0))
    pad = wid >= d
    stride = jnp.where(pad, jnp.int32(1), jnp.int32(H))
    c = jnp.where(pad, jnp.int32(0), c)
    lanemul = lanes * stride
    step16 = jnp.int32(LANES) * stride

    CH = T // NROUNDS
    bufs = (idx0, idx1)

    def fill(r, buf):
        bs0 = jnp.int32(r * CH) * stride + c

        @plsc.parallel_loop(0, CH // LANES, 1, unroll=UNROLL)
        def _(i):
            buf[pl.ds(i * LANES, LANES)] = (bs0 + i * step16) + lanemul

    @plsc.parallel_loop(0, HBINS, 1, unroll=UNROLL)
    def _(i):
        hist[pl.ds(i * LANES, LANES)] = jnp.zeros((LANES,), jnp.int32)

    @functools.partial(pl.run_scoped, sems=pltpu.SemaphoreType.DMA((2,)))
    def _(sems):
        def dma(r):
            return pltpu.make_async_copy(
                xflat_hbm.at[bufs[r % 2]],
                colbuf.at[pl.ds(r * CH, CH)],
                sems.at[r % 2],
            )

        fill(0, bufs[0])
        dma(0).start()
        fill(1, bufs[1])
        for r in range(NROUNDS):
            dma(r).wait()
            if r + 1 < NROUNDS:
                dma(r + 1).start()
            if r + 2 < NROUNDS:
                fill(r + 2, bufs[r % 2])
            pltpu.sync_copy(colbuf.at[pl.ds(r * CH, CH)],
                            xs_out.at[wid, pl.ds(r * CH, CH)])

            @plsc.parallel_loop(r * (CH // LANES), (r + 1) * (CH // LANES),
                                1, unroll=UNROLL)
            def _(i):
                idx = pl.ds(i * LANES, LANES)
                sk = _skey(colbuf[idx])
                colbuf[idx] = sk
                digit = (lax.shift_right_logical(sk, 32 - HB) ^ dflip) & dmask
                plsc.addupdate_scatter(hist, [laneoff + digit], ones)

    kk = jnp.int32(k)

    def scan(i, carry):
        run, bcnt, cmx = carry
        h = hist[pl.ds(i * LANES, LANES)]
        for l in range(1, LANES):
            h = h + hist[pl.ds(l * HBINS + i * LANES, LANES)]
        cum = run + plsc.cumsum(h)
        le = cum <= kk
        bcnt = bcnt + jnp.sum(jnp.where(le, 1, 0).astype(jnp.int32))
        cmx = jnp.maximum(cmx, jnp.max(jnp.where(le, cum, jnp.int32(0))))
        return jnp.max(cum), bcnt, cmx

    z = jnp.int32(0)
    _, b1, cmx = lax.fori_loop(0, HBINS // LANES, scan, (z, z, z))
    kk = kk - cmx

    @plsc.parallel_loop(0, nchunks, 1, unroll=UNROLL, carry=jnp.int32(0))
    def n2(i, off):
        sk = colbuf[pl.ds(i * LANES, LANES)]
        dig = (lax.shift_right_logical(sk, 32 - HB) ^ dflip) & dmask
        m = dig == b1
        mi = jnp.where(m, 1, 0).astype(jnp.int32)
        cs = plsc.cumsum(mi)
        plsc.store_scatter(colbuf, [off + cs - mi], sk, mask=m)
        return off + jnp.max(cs)

    sentinel = ((b1 ^ jnp.int32(1) ^ dflip) << (32 - HB))
    colbuf[pl.ds(n2, LANES)] = jnp.zeros((LANES,), jnp.int32) + sentinel
    nch2 = (n2 + jnp.int32(LANES - 1)) // jnp.int32(LANES)

    cur = b1
    for b in range(32 - HB - 1, -1, -1):
        pflip = jnp.int32(1 << (30 - b))
        curp = cur

        def cnt_body(i, acc, b=b, pflip=pflip, curp=curp):
            sk = colbuf[pl.ds(i * LANES, LANES)]
            pref = lax.shift_right_logical(sk, b + 1) ^ pflip
            hit = pref == curp
            bit0 = (lax.shift_right_logical(sk, b) & jnp.int32(1)) == 0
            m = jnp.logical_and(hit, bit0)
            return acc + jnp.sum(jnp.where(m, 1, 0).astype(jnp.int32))

        cnt0 = lax.fori_loop(0, nch2, cnt_body, jnp.int32(0))
        go1 = kk >= cnt0
        kk = jnp.where(go1, kk - cnt0, kk)
        cur = (cur << 1) | jnp.where(go1, jnp.int32(1), jnp.int32(0))

    medk = cur ^ SIGNBIT
    medbuf[...] = _skey(jnp.zeros((LANES,), jnp.int32) + medk)
    pltpu.sync_copy(medbuf, med_out.at[pl.ds(wid * LANES, LANES)])


def _bits_body(T, terms, slots, d, xs_hbm, med_hbm, mask_out, cnt_out,
               rowbuf, medv, maskbuf, cntrow):
    wid = lax.axis_index("s") * NC + lax.axis_index("c")
    RB = T // NW
    base = wid * RB
    for j in range(d):
        pltpu.sync_copy(xs_hbm.at[j, pl.ds(base, RB)],
                        rowbuf.at[pl.ds(j * RB, RB)])
    pltpu.sync_copy(med_hbm, medv)
    meds = [medv[pl.ds(j * LANES, LANES)] for j in range(d)]
    lanes = lax.iota(jnp.int32, LANES)

    @plsc.parallel_loop(0, RB // LANES, 1, unroll=4,
                        carry=jnp.zeros((LANES,), jnp.int32))
    def cnt(i, cnt):
        off = i * LANES
        bits = []
        for j in range(d):
            bits.append(rowbuf[pl.ds(j * RB + off, LANES)] > meds[j])
        bmask = jnp.zeros((LANES,), jnp.int32)
        for ci, (sel, sg) in enumerate(terms):
            c = ci + 1
            m = None
            for s, g in zip(sel, sg):
                t = bits[slots[s]] if g else jnp.logical_not(bits[slots[s]])
                m = t if m is None else jnp.logical_and(m, t)
            bmask = bmask + jnp.where(m, jnp.int32(1 << c), jnp.int32(0))
            pc = jnp.sum(jnp.where(m, 1, 0).astype(jnp.int32))
            cnt = cnt + jnp.where(lanes == c, pc, jnp.int32(0))
        maskbuf[pl.ds(off, LANES)] = bmask
        return cnt
    cntrow[...] = cnt
    pltpu.sync_copy(cntrow, cnt_out.at[pl.ds(wid * LANES, LANES)])
    pltpu.sync_copy(maskbuf, mask_out.at[pl.ds(base, RB)])


def _label_body(T, thresh, mask_hbm, cnt_hbm, out_hbm,
                maskbuf, cntv, lblbuf):
    wid = lax.axis_index("s") * NC + lax.axis_index("c")
    RB = T // NW
    base = wid * RB
    pltpu.sync_copy(cnt_hbm, cntv)
    tot = jnp.zeros((LANES,), jnp.int32)
    for i in range(NW):
        tot = tot + cntv[pl.ds(i * LANES, LANES)]
    lanes = lax.iota(jnp.int32, LANES)
    inrange = jnp.logical_and(lanes >= 1, lanes <= NCLS - 1)
    exceed = jnp.where(jnp.logical_and(tot > thresh, inrange), 1, 0)
    exceed = exceed.astype(jnp.int32)
    cs = plsc.cumsum(exceed)
    active = jnp.logical_and((cs - exceed) == 0, inrange)
    bitvals = lax.shift_left(jnp.int32(1), lanes)
    actmask = jnp.sum(jnp.where(active, bitvals, jnp.int32(0)))
    pltpu.sync_copy(mask_hbm.at[pl.ds(base, RB)], maskbuf)

    @plsc.parallel_loop(0, RB // LANES, 1, unroll=4)
    def _(i):
        idx = pl.ds(i * LANES, LANES)
        m = maskbuf[idx] & actmask
        lbl = jnp.zeros((LANES,), jnp.int32)
        for c in range(1, NCLS):
            hit = (lax.shift_right_logical(m, c) & 1) == 1
            lbl = jnp.where(hit, jnp.int32(c), lbl)
        lblbuf[idx] = lbl
    pltpu.sync_copy(lblbuf, out_hbm.at[pl.ds(base, RB)])


def kernel(input):
    x = input
    if x.ndim == 1:
        x = x[:, None]
    T, H = x.shape
    terms = _draw_terms(H)
    cols = sorted({s for sel, _ in terms for s in sel})
    slots = {c: i for i, c in enumerate(cols)}
    d = len(cols)
    cols_pad = cols + [cols[0]] * (NW - d)
    k = (T - 1) // 2
    thresh = T // (2 * NCLS)
    RB = T // NW

    xflat_i = lax.bitcast_convert_type(x.astype(jnp.float32),
                                       jnp.int32).reshape(-1)
    colidx = jnp.asarray(cols_pad, dtype=jnp.int32)

    mesh = plsc.VectorSubcoreMesh(core_axis_name="c", subcore_axis_name="s",
                                  num_cores=NC, num_subcores=NS)
    cparams = pltpu.CompilerParams(needs_layout_passes=False)

    med, xs_i = pl.kernel(
        functools.partial(_med_body, T, H, d, k),
        out_type=[jax.ShapeDtypeStruct((NW * LANES,), jnp.int32),
                  jax.ShapeDtypeStruct((NW, T), jnp.int32)],
        mesh=mesh,
        compiler_params=cparams,
        scratch_types=[
            pltpu.VMEM((T,), jnp.int32),
            pltpu.VMEM((LANES * HBINS,), jnp.int32),
            pltpu.VMEM((T // NROUNDS,), jnp.int32),
            pltpu.VMEM((T // NROUNDS,), jnp.int32),
            pltpu.VMEM((NW,), jnp.int32),
            pltpu.VMEM((LANES,), jnp.int32),
        ],
    )(xflat_i, colidx)
    med_f = lax.bitcast_convert_type(med, jnp.float32)
    xs_f = lax.bitcast_convert_type(xs_i, jnp.float32)

    bmask, cnts = pl.kernel(
        functools.partial(_bits_body, T, terms, slots, d),
        out_type=[jax.ShapeDtypeStruct((T,), jnp.int32),
                  jax.ShapeDtypeStruct((NW * LANES,), jnp.int32)],
        mesh=mesh,
        compiler_params=cparams,
        scratch_types=[
            pltpu.VMEM((d * RB,), jnp.float32),
            pltpu.VMEM((NW * LANES,), jnp.float32),
            pltpu.VMEM((RB,), jnp.int32),
            pltpu.VMEM((LANES,), jnp.int32),
        ],
    )(xs_f, med_f)

    out = pl.kernel(
        functools.partial(_label_body, T, thresh),
        out_type=jax.ShapeDtypeStruct((T,), jnp.int32),
        mesh=mesh,
        compiler_params=cparams,
        scratch_types=[
            pltpu.VMEM((RB,), jnp.int32),
            pltpu.VMEM((NW * LANES,), jnp.int32),
            pltpu.VMEM((RB,), jnp.int32),
        ],
    )(bmask, cnts)

    return out


# speedup vs baseline: 2.0491x; 2.0491x over previous
import functools

import numpy as np
import jax
import jax.numpy as jnp
from jax import lax
from jax.experimental import pallas as pl
from jax.experimental.pallas import tpu as pltpu
from jax.experimental.pallas import tpu_sc as plsc

NCLS = 10
NC = 2
NS = 16
NW = NC * NS
LANES = 16
UNROLL = 8
HB = 11
HBINS = 1 << HB
SIGNBIT = np.int32(-(1 << 31))


def _draw_terms(H):
    rng = np.random.default_rng(0)
    terms = []
    for _ in range(1, NCLS):
        _ = int(rng.integers(1, min(5, H) + 1))
        ts = int(rng.integers(1, min(3, H) + 1))
        sel = [int(v) for v in rng.integers(0, H, size=(ts,))]
        sg = [bool(v) for v in rng.integers(0, 2, size=(ts,))]
        terms.append((sel, sg))
    return terms


def _skey(bits):
    return bits ^ lax.shift_right_logical(
        lax.shift_right_arithmetic(bits, 31), 1)


def _med_body(T, k, xs_hbm, med_out, colbuf, hist, medbuf):
    wid = lax.axis_index("s") * NC + lax.axis_index("c")
    pltpu.sync_copy(xs_hbm.at[wid], colbuf)
    nchunks = T // LANES
    ones = jnp.ones((LANES,), jnp.int32)
    lanes = lax.iota(jnp.int32, LANES)
    laneoff = lanes * jnp.int32(HBINS)
    dflip = jnp.int32(1 << (HB - 1))
    dmask = jnp.int32(HBINS - 1)

    @plsc.parallel_loop(0, HBINS, 1, unroll=UNROLL)
    def _(i):
        hist[pl.ds(i * LANES, LANES)] = jnp.zeros((LANES,), jnp.int32)

    @plsc.parallel_loop(0, nchunks, 1, unroll=UNROLL)
    def _(i):
        idx = pl.ds(i * LANES, LANES)
        sk = _skey(colbuf[idx])
        colbuf[idx] = sk
        digit = (lax.shift_right_logical(sk, 32 - HB) ^ dflip) & dmask
        plsc.addupdate_scatter(hist, [laneoff + digit], ones)

    kk = jnp.int32(k)

    def scan(i, carry):
        run, bcnt, cmx = carry
        h = hist[pl.ds(i * LANES, LANES)]
        for l in range(1, LANES):
            h = h + hist[pl.ds(l * HBINS + i * LANES, LANES)]
        cum = run + plsc.cumsum(h)
        le = cum <= kk
        bcnt = bcnt + jnp.sum(jnp.where(le, 1, 0).astype(jnp.int32))
        cmx = jnp.maximum(cmx, jnp.max(jnp.where(le, cum, jnp.int32(0))))
        return jnp.max(cum), bcnt, cmx

    z = jnp.int32(0)
    _, b1, cmx = lax.fori_loop(0, HBINS // LANES, scan, (z, z, z))
    kk = kk - cmx

    @plsc.parallel_loop(0, nchunks, 1, unroll=UNROLL, carry=jnp.int32(0))
    def n2(i, off):
        sk = colbuf[pl.ds(i * LANES, LANES)]
        dig = (lax.shift_right_logical(sk, 32 - HB) ^ dflip) & dmask
        m = dig == b1
        mi = jnp.where(m, 1, 0).astype(jnp.int32)
        cs = plsc.cumsum(mi)
        plsc.store_scatter(colbuf, [off + cs - mi], sk, mask=m)
        return off + jnp.max(cs)

    sentinel = ((b1 ^ jnp.int32(1) ^ dflip) << (32 - HB))
    colbuf[pl.ds(n2, LANES)] = jnp.zeros((LANES,), jnp.int32) + sentinel
    nch2 = (n2 + jnp.int32(LANES - 1)) // jnp.int32(LANES)

    cur = b1
    for b in range(32 - HB - 1, -1, -1):
        pflip = jnp.int32(1 << (30 - b))
        curp = cur

        def cnt_body(i, acc, b=b, pflip=pflip, curp=curp):
            sk = colbuf[pl.ds(i * LANES, LANES)]
            pref = lax.shift_right_logical(sk, b + 1) ^ pflip
            hit = pref == curp
            bit0 = (lax.shift_right_logical(sk, b) & jnp.int32(1)) == 0
            m = jnp.logical_and(hit, bit0)
            return acc + jnp.sum(jnp.where(m, 1, 0).astype(jnp.int32))

        cnt0 = lax.fori_loop(0, nch2, cnt_body, jnp.int32(0))
        go1 = kk >= cnt0
        kk = jnp.where(go1, kk - cnt0, kk)
        cur = (cur << 1) | jnp.where(go1, jnp.int32(1), jnp.int32(0))

    medk = cur ^ SIGNBIT
    medbuf[...] = _skey(jnp.zeros((LANES,), jnp.int32) + medk)
    pltpu.sync_copy(medbuf, med_out.at[pl.ds(wid * LANES, LANES)])


def _bits_body(T, terms, slots, d, xs_hbm, med_hbm, mask_out, cnt_out,
               rowbuf, medv, maskbuf, cntrow):
    wid = lax.axis_index("s") * NC + lax.axis_index("c")
    RB = T // NW
    base = wid * RB
    for j in range(d):
        pltpu.sync_copy(xs_hbm.at[j, pl.ds(base, RB)],
                        rowbuf.at[pl.ds(j * RB, RB)])
    pltpu.sync_copy(med_hbm, medv)
    meds = [medv[pl.ds(j * LANES, LANES)] for j in range(d)]
    lanes = lax.iota(jnp.int32, LANES)

    @plsc.parallel_loop(0, RB // LANES, 1, unroll=4,
                        carry=jnp.zeros((LANES,), jnp.int32))
    def cnt(i, cnt):
        off = i * LANES
        bits = []
        for j in range(d):
            bits.append(rowbuf[pl.ds(j * RB + off, LANES)] > meds[j])
        bmask = jnp.zeros((LANES,), jnp.int32)
        for ci, (sel, sg) in enumerate(terms):
            c = ci + 1
            m = None
            for s, g in zip(sel, sg):
                t = bits[slots[s]] if g else jnp.logical_not(bits[slots[s]])
                m = t if m is None else jnp.logical_and(m, t)
            bmask = bmask + jnp.where(m, jnp.int32(1 << c), jnp.int32(0))
            pc = jnp.sum(jnp.where(m, 1, 0).astype(jnp.int32))
            cnt = cnt + jnp.where(lanes == c, pc, jnp.int32(0))
        maskbuf[pl.ds(off, LANES)] = bmask
        return cnt
    cntrow[...] = cnt
    pltpu.sync_copy(cntrow, cnt_out.at[pl.ds(wid * LANES, LANES)])
    pltpu.sync_copy(maskbuf, mask_out.at[pl.ds(base, RB)])


def _label_body(T, thresh, mask_hbm, cnt_hbm, out_hbm,
                maskbuf, cntv, lblbuf):
    wid = lax.axis_index("s") * NC + lax.axis_index("c")
    RB = T // NW
    base = wid * RB
    pltpu.sync_copy(cnt_hbm, cntv)
    tot = jnp.zeros((LANES,), jnp.int32)
    for i in range(NW):
        tot = tot + cntv[pl.ds(i * LANES, LANES)]
    lanes = lax.iota(jnp.int32, LANES)
    inrange = jnp.logical_and(lanes >= 1, lanes <= NCLS - 1)
    exceed = jnp.where(jnp.logical_and(tot > thresh, inrange), 1, 0)
    exceed = exceed.astype(jnp.int32)
    cs = plsc.cumsum(exceed)
    active = jnp.logical_and((cs - exceed) == 0, inrange)
    bitvals = lax.shift_left(jnp.int32(1), lanes)
    actmask = jnp.sum(jnp.where(active, bitvals, jnp.int32(0)))
    pltpu.sync_copy(mask_hbm.at[pl.ds(base, RB)], maskbuf)

    @plsc.parallel_loop(0, RB // LANES, 1, unroll=4)
    def _(i):
        idx = pl.ds(i * LANES, LANES)
        m = maskbuf[idx] & actmask
        lbl = jnp.zeros((LANES,), jnp.int32)
        for c in range(1, NCLS):
            hit = (lax.shift_right_logical(m, c) & 1) == 1
            lbl = jnp.where(hit, jnp.int32(c), lbl)
        lblbuf[idx] = lbl
    pltpu.sync_copy(lblbuf, out_hbm.at[pl.ds(base, RB)])


def kernel(input):
    x = input
    if x.ndim == 1:
        x = x[:, None]
    T, H = x.shape
    terms = _draw_terms(H)
    cols = sorted({s for sel, _ in terms for s in sel})
    slots = {c: i for i, c in enumerate(cols)}
    d = len(cols)
    cols_pad = cols + [cols[0]] * (NW - d)
    k = (T - 1) // 2
    thresh = T // (2 * NCLS)
    RB = T // NW

    xs_f = jnp.take(x.astype(jnp.float32),
                    jnp.asarray(cols_pad, dtype=jnp.int32), axis=1).T
    xs_i = lax.bitcast_convert_type(xs_f, jnp.int32)

    mesh = plsc.VectorSubcoreMesh(core_axis_name="c", subcore_axis_name="s",
                                  num_cores=NC, num_subcores=NS)
    cparams = pltpu.CompilerParams(needs_layout_passes=False)

    med = pl.kernel(
        functools.partial(_med_body, T, k),
        out_type=jax.ShapeDtypeStruct((NW * LANES,), jnp.int32),
        mesh=mesh,
        compiler_params=cparams,
        scratch_types=[
            pltpu.VMEM((T,), jnp.int32),
            pltpu.VMEM((LANES * HBINS,), jnp.int32),
            pltpu.VMEM((LANES,), jnp.int32),
        ],
    )(xs_i)
    med_f = lax.bitcast_convert_type(med, jnp.float32)

    bmask, cnts = pl.kernel(
        functools.partial(_bits_body, T, terms, slots, d),
        out_type=[jax.ShapeDtypeStruct((T,), jnp.int32),
                  jax.ShapeDtypeStruct((NW * LANES,), jnp.int32)],
        mesh=mesh,
        compiler_params=cparams,
        scratch_types=[
            pltpu.VMEM((d * RB,), jnp.float32),
            pltpu.VMEM((NW * LANES,), jnp.float32),
            pltpu.VMEM((RB,), jnp.int32),
            pltpu.VMEM((LANES,), jnp.int32),
        ],
    )(xs_f, med_f)

    out = pl.kernel(
        functools.partial(_label_body, T, thresh),
        out_type=jax.ShapeDtypeStruct((T,), jnp.int32),
        mesh=mesh,
        compiler_params=cparams,
        scratch_types=[
            pltpu.VMEM((RB,), jnp.int32),
            pltpu.VMEM((NW * LANES,), jnp.int32),
            pltpu.VMEM((RB,), jnp.int32),
        ],
    )(bmask, cnts)

    return out
